# baseline (speedup 1.0000x reference)
import math

import jax
import jax.numpy as jnp
from jax.experimental import pallas as pl
from jax.experimental.pallas import tpu as pltpu

DECAY = 0.99
TOP_K = 8


def _attn_kernel(q_ref, wq_ref, wk_ref, imp_ref, age_ref, mem_ref,
                 attn_ref, ret_ref, s_ref, w_ref, kp_ref, bias_ref):
    tl = q_ref.shape[1]
    d = q_ref.shape[-1]

    @pl.when(pl.program_id(0) + pl.program_id(1) == 0)
    def _proj():
        kp_ref[...] = jnp.dot(mem_ref[...], wk_ref[...],
                              preferred_element_type=jnp.float32)
        eff = imp_ref[...] * jnp.exp(age_ref[...] * math.log(DECAY))
        bias_ref[...] = jnp.maximum(jnp.log(eff), -10.0)

    qp = jnp.dot(q_ref[0], wq_ref[...], preferred_element_type=jnp.float32)
    s = jax.lax.dot_general(qp, kp_ref[...], (((1,), (1,)), ((), ())),
                            preferred_element_type=jnp.float32)
    s = s * (1.0 / math.sqrt(d)) + bias_ref[...]
    s_ref[...] = s

    n_slots = s.shape[-1]
    neg_inf = jnp.float32(-jnp.inf)

    work = s
    m0 = None
    for i in range(TOP_K):
        m = jnp.max(work, axis=1, keepdims=True)
        if i == 0:
            m0 = m
        work = jnp.where(work == m, neg_inf, work)
    w_ref[...] = work
    n_sel = jnp.sum((work == neg_inf).astype(jnp.float32))

    @pl.when(n_sel != float(TOP_K * tl))
    def _exact_repair():
        iota = jax.lax.broadcasted_iota(jnp.int32, (tl, n_slots), 1)
        work2 = s_ref[...]
        for _ in range(TOP_K):
            m = jnp.max(work2, axis=1, keepdims=True)
            first = jnp.min(jnp.where(work2 == m, iota, n_slots), axis=1,
                            keepdims=True)
            work2 = jnp.where(iota == first, neg_inf, work2)
        w_ref[...] = work2

    sel = w_ref[...] == neg_inf
    e = jnp.where(sel, jnp.exp(s_ref[...] - m0), 0.0)
    attn = e / jnp.sum(e, axis=1, keepdims=True)
    attn_ref[0] = attn
    ret_ref[0] = jnp.dot(attn, mem_ref[...],
                         preferred_element_type=jnp.float32)


def kernel(query, memory, importance, age, W_q, W_k, top_k):
    B, L, d = query.shape
    S = memory.shape[1]
    mem2d = memory.reshape(S, d)
    tl = min(2048, L)
    grid = (B, L // tl)
    attn, ret = pl.pallas_call(
        _attn_kernel,
        grid=grid,
        in_specs=[
            pl.BlockSpec((1, tl, d), lambda b, l: (b, l, 0)),
            pl.BlockSpec((d, d), lambda b, l: (0, 0)),
            pl.BlockSpec((d, d), lambda b, l: (0, 0)),
            pl.BlockSpec((1, S), lambda b, l: (0, 0)),
            pl.BlockSpec((1, S), lambda b, l: (0, 0)),
            pl.BlockSpec((S, d), lambda b, l: (0, 0)),
        ],
        out_specs=[
            pl.BlockSpec((1, tl, S), lambda b, l: (b, l, 0)),
            pl.BlockSpec((1, tl, d), lambda b, l: (b, l, 0)),
        ],
        out_shape=[
            jax.ShapeDtypeStruct((B, L, S), jnp.float32),
            jax.ShapeDtypeStruct((B, L, d), jnp.float32),
        ],
        scratch_shapes=[
            pltpu.VMEM((tl, S), jnp.float32),
            pltpu.VMEM((tl, S), jnp.float32),
            pltpu.VMEM((S, d), jnp.float32),
            pltpu.VMEM((1, S), jnp.float32),
        ],
        compiler_params=pltpu.CompilerParams(
            dimension_semantics=("arbitrary", "arbitrary")),
    )(query, W_q, W_k, importance, age, mem2d)
    return ret, attn
